# Initial kernel scaffold; baseline (speedup 1.0000x reference)
import jax
import jax.numpy as jnp
from jax.experimental import pallas as pl


def kernel(x, router_w, router_b, W1, B1, W2, B2, SW1, SB1, SW2, SB2):
    raise NotImplementedError("write your pallas kernel here")


# speedup vs baseline: 6.7458x; 6.7458x over previous
import functools

import jax
import jax.numpy as jnp
from jax import lax
from jax.experimental import pallas as pl
from jax.experimental.pallas import tpu as pltpu
from jax.experimental.pallas import tpu_sc as plsc

N = 2048
H = 768
FFN = 3072
E = 8
CAP = 320
FB_CAP = 2240
NSLOT = E * CAP + FB_CAP
CH = 256
NW = 32
TPW = N // NW


def _router_body(x_ref, w_ref, b_ref,
                 logits_ref, gates_ref, idx_ref, slot_ref, nfb_ref,
                 util_ref, f_ref, p_ref):
    x = x_ref[...]
    logits = jnp.dot(x, w_ref[...], preferred_element_type=jnp.float32) + b_ref[...]
    logits_ref[...] = logits

    m = jnp.max(logits, axis=1, keepdims=True)
    ex = jnp.exp(logits - m)
    g = ex / jnp.sum(ex, axis=1, keepdims=True)

    lane = lax.broadcasted_iota(jnp.int32, (N, E), 1)
    m1 = jnp.max(g, axis=1, keepdims=True)
    i1 = jnp.min(jnp.where(g == m1, lane, E), axis=1, keepdims=True)
    g2m = jnp.where(lane == i1, -1.0, g)
    m2 = jnp.max(g2m, axis=1, keepdims=True)
    i2 = jnp.min(jnp.where(g2m == m2, lane, E), axis=1, keepdims=True)

    topmask = (lane == i1) | (lane == i2)
    gates = jnp.where(topmask, g, 0.0) / (m1 + m2 + 1e-9)
    gates_ref[...] = gates
    idx_ref[...] = jnp.concatenate([i1, i2], axis=1)
    p_ref[...] = jnp.sum(gates, axis=0, keepdims=True) * (1.0 / N)

    r = lax.broadcasted_iota(jnp.int32, (CH, CH), 0)
    c = lax.broadcasted_iota(jnp.int32, (CH, CH), 1)
    L = (r >= c).astype(jnp.float32)

    def cumsum_tokens(oh):
        parts = []
        carry = jnp.zeros((1, oh.shape[1]), jnp.float32)
        for ci in range(N // CH):
            cs = jnp.dot(L, oh[ci * CH:(ci + 1) * CH],
                         preferred_element_type=jnp.float32) + carry
            carry = cs[CH - 1:CH, :]
            parts.append(cs)
        return jnp.concatenate(parts, axis=0)

    oh_p = (lane == i1).astype(jnp.float32)
    rp = cumsum_tokens(oh_p)
    counts_p = rp[N - 1:N, :]
    tok_rank = jnp.sum(rp * oh_p, axis=1, keepdims=True) - 1.0
    keep = tok_rank < CAP
    used = jnp.minimum(counts_p, float(CAP))
    overflow = ~keep

    oh_s = ((lane == i2) & overflow).astype(jnp.float32)
    rs = cumsum_tokens(oh_s)
    tok_rank_s = jnp.sum(rs * oh_s, axis=1, keepdims=True) - 1.0
    free_tok = jnp.sum(oh_s * (float(CAP) - used), axis=1, keepdims=True)
    used_tok = jnp.sum(oh_s * used, axis=1, keepdims=True)
    take_second = overflow & (tok_rank_s < free_tok)
    use_fb = overflow & (~take_second)

    fb_cum = cumsum_tokens(use_fb.astype(jnp.float32))
    nfb_ref[...] = fb_cum[N - 1:N, :].astype(jnp.int32)

    slot = jnp.where(keep, i1.astype(jnp.float32) * CAP + tok_rank,
                     jnp.where(take_second,
                               i2.astype(jnp.float32) * CAP + used_tok + tok_rank_s,
                               E * CAP + fb_cum - 1.0))
    slot_ref[...] = slot.astype(jnp.int32)

    oh_a = oh_p * keep.astype(jnp.float32) \
        + ((lane == i2) & take_second).astype(jnp.float32)
    util_ref[...] = jnp.sum(oh_a, axis=0, keepdims=True) * (1.0 / N)
    f_ref[...] = counts_p * (1.0 / N)


_router_call = pl.pallas_call(
    _router_body,
    out_shape=(
        jax.ShapeDtypeStruct((N, E), jnp.float32),
        jax.ShapeDtypeStruct((N, E), jnp.float32),
        jax.ShapeDtypeStruct((N, 2), jnp.int32),
        jax.ShapeDtypeStruct((N, 1), jnp.int32),
        jax.ShapeDtypeStruct((1, 1), jnp.int32),
        jax.ShapeDtypeStruct((1, E), jnp.float32),
        jax.ShapeDtypeStruct((1, E), jnp.float32),
        jax.ShapeDtypeStruct((1, E), jnp.float32),
    ),
)


_sc_mesh = plsc.VectorSubcoreMesh(core_axis_name="c", subcore_axis_name="s")


@functools.partial(
    pl.kernel, mesh=_sc_mesh,
    out_type=jax.ShapeDtypeStruct((NSLOT, H), jnp.float32),
    scratch_types=[
        pltpu.VMEM((TPW,), jnp.int32),
        pltpu.VMEM((TPW, H), jnp.float32),
        pltpu.SemaphoreType.DMA,
    ],
)
def _dispatch(x_hbm, slot_hbm, buf_hbm, idx_v, rows_v, sem):
    wid = lax.axis_index("s") * 2 + lax.axis_index("c")
    base = wid * TPW
    pltpu.sync_copy(slot_hbm.at[pl.ds(base, TPW)], idx_v)
    pltpu.sync_copy(x_hbm.at[pl.ds(base, TPW)], rows_v)
    pltpu.async_copy(rows_v, buf_hbm.at[idx_v], sem).wait()


@functools.partial(
    pl.kernel, mesh=_sc_mesh,
    out_type=jax.ShapeDtypeStruct((N, H), jnp.float32),
    scratch_types=[
        pltpu.VMEM((TPW,), jnp.int32),
        pltpu.VMEM((TPW, H), jnp.float32),
        pltpu.SemaphoreType.DMA,
    ],
)
def _combine(buf_hbm, slot_hbm, y_hbm, idx_v, rows_v, sem):
    wid = lax.axis_index("s") * 2 + lax.axis_index("c")
    base = wid * TPW
    pltpu.sync_copy(slot_hbm.at[pl.ds(base, TPW)], idx_v)
    pltpu.async_copy(buf_hbm.at[idx_v], rows_v, sem).wait()
    pltpu.sync_copy(rows_v, y_hbm.at[pl.ds(base, TPW)])


def _gelu_exact(x):
    return 0.5 * x * (1.0 + jnp.tanh(0.7978845608028654 * (x + 0.044715 * x * x * x)))


def _ffn_block(xb, w1, b1, w2, b2):
    T = 4
    C = FFN // T
    acc = jnp.broadcast_to(b2, (CAP, H))
    for t in range(T):
        h = jnp.dot(xb, w1[:, t * C:(t + 1) * C],
                    preferred_element_type=jnp.float32) + b1[:, t * C:(t + 1) * C]
        g = _gelu_exact(h)
        acc = acc + jnp.dot(g, w2[t * C:(t + 1) * C, :],
                            preferred_element_type=jnp.float32)
    return acc


def _ffn_body(nfb_ref, buf_ref, w1_ref, b1_ref, w2_ref, b2_ref,
              sw1_ref, sb1_ref, sw2_ref, sb2_ref, out_ref):
    i = pl.program_id(0)
    xb = buf_ref[...]

    @pl.when(i < E)
    def _expert():
        out_ref[...] = _ffn_block(xb, w1_ref[0], b1_ref[0], w2_ref[0], b2_ref[0])

    @pl.when((i >= E) & ((i - E) * CAP < nfb_ref[0]))
    def _fallback():
        out_ref[...] = _ffn_block(xb, sw1_ref[...], sb1_ref[...],
                                  sw2_ref[...], sb2_ref[...])


_ffn_call = pl.pallas_call(
    _ffn_body,
    grid_spec=pltpu.PrefetchScalarGridSpec(
        num_scalar_prefetch=1,
        grid=(NSLOT // CAP,),
        in_specs=[
            pl.BlockSpec((CAP, H), lambda i, n: (i, 0)),
            pl.BlockSpec((1, H, FFN), lambda i, n: (jnp.minimum(i, E - 1), 0, 0)),
            pl.BlockSpec((1, 1, FFN), lambda i, n: (jnp.minimum(i, E - 1), 0, 0)),
            pl.BlockSpec((1, FFN, H), lambda i, n: (jnp.minimum(i, E - 1), 0, 0)),
            pl.BlockSpec((1, 1, H), lambda i, n: (jnp.minimum(i, E - 1), 0, 0)),
            pl.BlockSpec((H, FFN), lambda i, n: (0, 0)),
            pl.BlockSpec((1, FFN), lambda i, n: (0, 0)),
            pl.BlockSpec((FFN, H), lambda i, n: (0, 0)),
            pl.BlockSpec((1, H), lambda i, n: (0, 0)),
        ],
        out_specs=pl.BlockSpec((CAP, H), lambda i, n: (i, 0)),
    ),
    out_shape=jax.ShapeDtypeStruct((NSLOT, H), jnp.float32),
    compiler_params=pltpu.CompilerParams(vmem_limit_bytes=100 * 1024 * 1024),
)


def kernel(x, router_w, router_b, W1, B1, W2, B2, SW1, SB1, SW2, SB2):
    x_f = x.reshape(N, H)
    (logits, gates, idx, slot, nfb, util, f, P) = _router_call(
        x_f, router_w, router_b.reshape(1, E))
    slot_flat = slot.reshape(N)
    buf = _dispatch(x_f, slot_flat)
    out_buf = _ffn_call(nfb.reshape(1), buf,
                        W1, B1.reshape(E, 1, FFN), W2, B2.reshape(E, 1, H),
                        SW1, SB1.reshape(1, FFN), SW2, SB2.reshape(1, H))
    y_f = _combine(out_buf, slot_flat)
    return (y_f.reshape(x.shape), util.reshape(E), f.reshape(E), P.reshape(E),
            logits, gates, idx)


# speedup vs baseline: 6.7630x; 1.0025x over previous
import functools

import jax
import jax.numpy as jnp
from jax import lax
from jax.experimental import pallas as pl
from jax.experimental.pallas import tpu as pltpu
from jax.experimental.pallas import tpu_sc as plsc

N = 2048
H = 768
FFN = 3072
E = 8
CAP = 320
FB_CAP = 2240
NSLOT = E * CAP + FB_CAP
CH = 256
NW = 32
TPW = N // NW


def _router_body(x_ref, w_ref, b_ref,
                 logits_ref, gates_ref, idx_ref, slot_ref, nfb_ref,
                 util_ref, f_ref, p_ref):
    x = x_ref[...]
    logits = jnp.dot(x, w_ref[...], preferred_element_type=jnp.float32) + b_ref[...]
    logits_ref[...] = logits

    m = jnp.max(logits, axis=1, keepdims=True)
    ex = jnp.exp(logits - m)
    g = ex / jnp.sum(ex, axis=1, keepdims=True)

    lane = lax.broadcasted_iota(jnp.int32, (N, E), 1)
    m1 = jnp.max(g, axis=1, keepdims=True)
    i1 = jnp.min(jnp.where(g == m1, lane, E), axis=1, keepdims=True)
    g2m = jnp.where(lane == i1, -1.0, g)
    m2 = jnp.max(g2m, axis=1, keepdims=True)
    i2 = jnp.min(jnp.where(g2m == m2, lane, E), axis=1, keepdims=True)

    topmask = (lane == i1) | (lane == i2)
    gates = jnp.where(topmask, g, 0.0) / (m1 + m2 + 1e-9)
    gates_ref[...] = gates
    idx_ref[...] = jnp.concatenate([i1, i2], axis=1)
    p_ref[...] = jnp.sum(gates, axis=0, keepdims=True) * (1.0 / N)

    r = lax.broadcasted_iota(jnp.int32, (CH, CH), 0)
    c = lax.broadcasted_iota(jnp.int32, (CH, CH), 1)
    L = (r >= c).astype(jnp.float32)

    def cumsum_tokens(oh):
        parts = []
        carry = jnp.zeros((1, oh.shape[1]), jnp.float32)
        for ci in range(N // CH):
            cs = jnp.dot(L, oh[ci * CH:(ci + 1) * CH],
                         preferred_element_type=jnp.float32) + carry
            carry = cs[CH - 1:CH, :]
            parts.append(cs)
        return jnp.concatenate(parts, axis=0)

    oh_p = (lane == i1).astype(jnp.float32)
    rp = cumsum_tokens(oh_p)
    counts_p = rp[N - 1:N, :]
    tok_rank = jnp.sum(rp * oh_p, axis=1, keepdims=True) - 1.0
    keep = tok_rank < CAP
    used = jnp.minimum(counts_p, float(CAP))
    overflow = ~keep

    oh_s = ((lane == i2) & overflow).astype(jnp.float32)
    rs = cumsum_tokens(oh_s)
    tok_rank_s = jnp.sum(rs * oh_s, axis=1, keepdims=True) - 1.0
    free_tok = jnp.sum(oh_s * (float(CAP) - used), axis=1, keepdims=True)
    used_tok = jnp.sum(oh_s * used, axis=1, keepdims=True)
    take_second = overflow & (tok_rank_s < free_tok)
    use_fb = overflow & (~take_second)

    fb_cum = cumsum_tokens(use_fb.astype(jnp.float32))
    nfb_ref[...] = fb_cum[N - 1:N, :].astype(jnp.int32)

    slot = jnp.where(keep, i1.astype(jnp.float32) * CAP + tok_rank,
                     jnp.where(take_second,
                               i2.astype(jnp.float32) * CAP + used_tok + tok_rank_s,
                               E * CAP + fb_cum - 1.0))
    slot_ref[...] = slot.astype(jnp.int32)

    oh_a = oh_p * keep.astype(jnp.float32) \
        + ((lane == i2) & take_second).astype(jnp.float32)
    util_ref[...] = jnp.sum(oh_a, axis=0, keepdims=True) * (1.0 / N)
    f_ref[...] = counts_p * (1.0 / N)


_router_call = pl.pallas_call(
    _router_body,
    out_shape=(
        jax.ShapeDtypeStruct((N, E), jnp.float32),
        jax.ShapeDtypeStruct((N, E), jnp.float32),
        jax.ShapeDtypeStruct((N, 2), jnp.int32),
        jax.ShapeDtypeStruct((N, 1), jnp.int32),
        jax.ShapeDtypeStruct((1, 1), jnp.int32),
        jax.ShapeDtypeStruct((1, E), jnp.float32),
        jax.ShapeDtypeStruct((1, E), jnp.float32),
        jax.ShapeDtypeStruct((1, E), jnp.float32),
    ),
)


_sc_mesh = plsc.VectorSubcoreMesh(core_axis_name="c", subcore_axis_name="s")


@functools.partial(
    pl.kernel, mesh=_sc_mesh,
    out_type=jax.ShapeDtypeStruct((NSLOT, H), jnp.float32),
    scratch_types=[
        pltpu.VMEM((TPW,), jnp.int32),
        pltpu.VMEM((TPW, H), jnp.float32),
        pltpu.SemaphoreType.DMA,
    ],
)
def _dispatch(x_hbm, slot_hbm, buf_hbm, idx_v, rows_v, sem):
    wid = lax.axis_index("s") * 2 + lax.axis_index("c")
    base = wid * TPW
    pltpu.sync_copy(slot_hbm.at[pl.ds(base, TPW)], idx_v)
    pltpu.sync_copy(x_hbm.at[pl.ds(base, TPW)], rows_v)
    pltpu.async_copy(rows_v, buf_hbm.at[idx_v], sem).wait()


@functools.partial(
    pl.kernel, mesh=_sc_mesh,
    out_type=jax.ShapeDtypeStruct((N, H), jnp.float32),
    scratch_types=[
        pltpu.VMEM((TPW,), jnp.int32),
        pltpu.VMEM((TPW, H), jnp.float32),
        pltpu.SemaphoreType.DMA,
    ],
)
def _combine(buf_hbm, slot_hbm, y_hbm, idx_v, rows_v, sem):
    wid = lax.axis_index("s") * 2 + lax.axis_index("c")
    base = wid * TPW
    pltpu.sync_copy(slot_hbm.at[pl.ds(base, TPW)], idx_v)
    pltpu.async_copy(buf_hbm.at[idx_v], rows_v, sem).wait()
    pltpu.sync_copy(rows_v, y_hbm.at[pl.ds(base, TPW)])


def _gelu_exact(x):
    return 0.5 * x * (1.0 + jnp.tanh(0.7978845608028654 * (x + 0.044715 * x * x * x)))


def _ffn_block(xb, w1, b1, w2, b2):
    T = 4
    C = FFN // T
    xb16 = xb.astype(jnp.bfloat16)
    acc = jnp.broadcast_to(b2, (CAP, H))
    for t in range(T):
        h = jnp.dot(xb16, w1[:, t * C:(t + 1) * C].astype(jnp.bfloat16),
                    preferred_element_type=jnp.float32) + b1[:, t * C:(t + 1) * C]
        g = _gelu_exact(h).astype(jnp.bfloat16)
        acc = acc + jnp.dot(g, w2[t * C:(t + 1) * C, :].astype(jnp.bfloat16),
                            preferred_element_type=jnp.float32)
    return acc


def _ffn_body(nfb_ref, buf_ref, w1_ref, b1_ref, w2_ref, b2_ref,
              sw1_ref, sb1_ref, sw2_ref, sb2_ref, out_ref):
    i = pl.program_id(0)
    xb = buf_ref[...]

    @pl.when(i < E)
    def _expert():
        out_ref[...] = _ffn_block(xb, w1_ref[0], b1_ref[0], w2_ref[0], b2_ref[0])

    @pl.when((i >= E) & ((i - E) * CAP < nfb_ref[0]))
    def _fallback():
        out_ref[...] = _ffn_block(xb, sw1_ref[...], sb1_ref[...],
                                  sw2_ref[...], sb2_ref[...])


_ffn_call = pl.pallas_call(
    _ffn_body,
    grid_spec=pltpu.PrefetchScalarGridSpec(
        num_scalar_prefetch=1,
        grid=(NSLOT // CAP,),
        in_specs=[
            pl.BlockSpec((CAP, H), lambda i, n: (i, 0)),
            pl.BlockSpec((1, H, FFN), lambda i, n: (jnp.minimum(i, E - 1), 0, 0)),
            pl.BlockSpec((1, 1, FFN), lambda i, n: (jnp.minimum(i, E - 1), 0, 0)),
            pl.BlockSpec((1, FFN, H), lambda i, n: (jnp.minimum(i, E - 1), 0, 0)),
            pl.BlockSpec((1, 1, H), lambda i, n: (jnp.minimum(i, E - 1), 0, 0)),
            pl.BlockSpec((H, FFN), lambda i, n: (0, 0)),
            pl.BlockSpec((1, FFN), lambda i, n: (0, 0)),
            pl.BlockSpec((FFN, H), lambda i, n: (0, 0)),
            pl.BlockSpec((1, H), lambda i, n: (0, 0)),
        ],
        out_specs=pl.BlockSpec((CAP, H), lambda i, n: (i, 0)),
    ),
    out_shape=jax.ShapeDtypeStruct((NSLOT, H), jnp.float32),
    compiler_params=pltpu.CompilerParams(vmem_limit_bytes=100 * 1024 * 1024),
)


def kernel(x, router_w, router_b, W1, B1, W2, B2, SW1, SB1, SW2, SB2):
    x_f = x.reshape(N, H)
    (logits, gates, idx, slot, nfb, util, f, P) = _router_call(
        x_f, router_w, router_b.reshape(1, E))
    slot_flat = slot.reshape(N)
    buf = _dispatch(x_f, slot_flat)
    out_buf = _ffn_call(nfb.reshape(1), buf,
                        W1, B1.reshape(E, 1, FFN), W2, B2.reshape(E, 1, H),
                        SW1, SB1.reshape(1, FFN), SW2, SB2.reshape(1, H))
    y_f = _combine(out_buf, slot_flat)
    return (y_f.reshape(x.shape), util.reshape(E), f.reshape(E), P.reshape(E),
            logits, gates, idx)


# speedup vs baseline: 6.8737x; 1.0164x over previous
import functools

import jax
import jax.numpy as jnp
from jax import lax
from jax.experimental import pallas as pl
from jax.experimental.pallas import tpu as pltpu
from jax.experimental.pallas import tpu_sc as plsc

N = 2048
H = 768
FFN = 3072
E = 8
CAP = 320
FB_CAP = 1600
NSLOT = E * CAP + FB_CAP
CH = 256
NW = 32
TPW = N // NW


def _router_body(x_ref, w_ref, b_ref,
                 logits_ref, gates_ref, idx_ref, slot_ref, nfb_ref,
                 util_ref, f_ref, p_ref):
    x = x_ref[...]
    logits = jnp.dot(x, w_ref[...], preferred_element_type=jnp.float32) + b_ref[...]
    logits_ref[...] = logits

    m = jnp.max(logits, axis=1, keepdims=True)
    ex = jnp.exp(logits - m)
    g = ex / jnp.sum(ex, axis=1, keepdims=True)

    lane = lax.broadcasted_iota(jnp.int32, (N, E), 1)
    m1 = jnp.max(g, axis=1, keepdims=True)
    i1 = jnp.min(jnp.where(g == m1, lane, E), axis=1, keepdims=True)
    g2m = jnp.where(lane == i1, -1.0, g)
    m2 = jnp.max(g2m, axis=1, keepdims=True)
    i2 = jnp.min(jnp.where(g2m == m2, lane, E), axis=1, keepdims=True)

    topmask = (lane == i1) | (lane == i2)
    gates = jnp.where(topmask, g, 0.0) / (m1 + m2 + 1e-9)
    gates_ref[...] = gates
    idx_ref[...] = jnp.concatenate([i1, i2], axis=1)
    p_ref[...] = jnp.sum(gates, axis=0, keepdims=True) * (1.0 / N)

    r = lax.broadcasted_iota(jnp.int32, (CH, CH), 0)
    c = lax.broadcasted_iota(jnp.int32, (CH, CH), 1)
    L = (r >= c).astype(jnp.float32)

    def cumsum_tokens(oh):
        parts = []
        carry = jnp.zeros((1, oh.shape[1]), jnp.float32)
        for ci in range(N // CH):
            cs = jnp.dot(L, oh[ci * CH:(ci + 1) * CH],
                         preferred_element_type=jnp.float32) + carry
            carry = cs[CH - 1:CH, :]
            parts.append(cs)
        return jnp.concatenate(parts, axis=0)

    oh_p = (lane == i1).astype(jnp.float32)
    rp = cumsum_tokens(oh_p)
    counts_p = rp[N - 1:N, :]
    tok_rank = jnp.sum(rp * oh_p, axis=1, keepdims=True) - 1.0
    keep = tok_rank < CAP
    used = jnp.minimum(counts_p, float(CAP))
    overflow = ~keep

    oh_s = ((lane == i2) & overflow).astype(jnp.float32)
    rs = cumsum_tokens(oh_s)
    tok_rank_s = jnp.sum(rs * oh_s, axis=1, keepdims=True) - 1.0
    free_tok = jnp.sum(oh_s * (float(CAP) - used), axis=1, keepdims=True)
    used_tok = jnp.sum(oh_s * used, axis=1, keepdims=True)
    take_second = overflow & (tok_rank_s < free_tok)
    use_fb = overflow & (~take_second)

    fb_cum = cumsum_tokens(use_fb.astype(jnp.float32))
    nfb_ref[...] = fb_cum[N - 1:N, :].astype(jnp.int32)

    slot = jnp.where(keep, i1.astype(jnp.float32) * CAP + tok_rank,
                     jnp.where(take_second,
                               i2.astype(jnp.float32) * CAP + used_tok + tok_rank_s,
                               E * CAP + fb_cum - 1.0))
    slot_ref[...] = slot.astype(jnp.int32)

    oh_a = oh_p * keep.astype(jnp.float32) \
        + ((lane == i2) & take_second).astype(jnp.float32)
    util_ref[...] = jnp.sum(oh_a, axis=0, keepdims=True) * (1.0 / N)
    f_ref[...] = counts_p * (1.0 / N)


_router_call = pl.pallas_call(
    _router_body,
    out_shape=(
        jax.ShapeDtypeStruct((N, E), jnp.float32),
        jax.ShapeDtypeStruct((N, E), jnp.float32),
        jax.ShapeDtypeStruct((N, 2), jnp.int32),
        jax.ShapeDtypeStruct((N, 1), jnp.int32),
        jax.ShapeDtypeStruct((1, 1), jnp.int32),
        jax.ShapeDtypeStruct((1, E), jnp.float32),
        jax.ShapeDtypeStruct((1, E), jnp.float32),
        jax.ShapeDtypeStruct((1, E), jnp.float32),
    ),
)


_sc_mesh = plsc.VectorSubcoreMesh(core_axis_name="c", subcore_axis_name="s")


@functools.partial(
    pl.kernel, mesh=_sc_mesh,
    out_type=jax.ShapeDtypeStruct((NSLOT, H), jnp.float32),
    scratch_types=[
        pltpu.VMEM((TPW,), jnp.int32),
        pltpu.VMEM((TPW, H), jnp.float32),
        pltpu.SemaphoreType.DMA,
    ],
)
def _dispatch(x_hbm, slot_hbm, buf_hbm, idx_v, rows_v, sem):
    wid = lax.axis_index("s") * 2 + lax.axis_index("c")
    base = wid * TPW
    pltpu.sync_copy(slot_hbm.at[pl.ds(base, TPW)], idx_v)
    pltpu.sync_copy(x_hbm.at[pl.ds(base, TPW)], rows_v)
    pltpu.async_copy(rows_v, buf_hbm.at[idx_v], sem).wait()


@functools.partial(
    pl.kernel, mesh=_sc_mesh,
    out_type=jax.ShapeDtypeStruct((N, H), jnp.float32),
    scratch_types=[
        pltpu.VMEM((TPW,), jnp.int32),
        pltpu.VMEM((TPW, H), jnp.float32),
        pltpu.SemaphoreType.DMA,
    ],
)
def _combine(buf_hbm, slot_hbm, y_hbm, idx_v, rows_v, sem):
    wid = lax.axis_index("s") * 2 + lax.axis_index("c")
    base = wid * TPW
    pltpu.sync_copy(slot_hbm.at[pl.ds(base, TPW)], idx_v)
    pltpu.async_copy(buf_hbm.at[idx_v], rows_v, sem).wait()
    pltpu.sync_copy(rows_v, y_hbm.at[pl.ds(base, TPW)])


def _gelu_exact(x):
    return 0.5 * x * (1.0 + jnp.tanh(0.7978845608028654 * (x + 0.044715 * x * x * x)))


def _ffn_block(xb, w1, b1, w2, b2):
    T = 4
    C = FFN // T
    acc = jnp.broadcast_to(b2, (CAP, H))
    for t in range(T):
        h = jnp.dot(xb, w1[:, t * C:(t + 1) * C],
                    preferred_element_type=jnp.float32) + b1[:, t * C:(t + 1) * C]
        g = _gelu_exact(h)
        acc = acc + jnp.dot(g, w2[t * C:(t + 1) * C, :],
                            preferred_element_type=jnp.float32)
    return acc


def _ffn_body(nfb_ref, buf_ref, w1_ref, b1_ref, w2_ref, b2_ref,
              sw1_ref, sb1_ref, sw2_ref, sb2_ref, out_ref):
    i = pl.program_id(0)
    xb = buf_ref[...]

    @pl.when(i < E)
    def _expert():
        out_ref[...] = _ffn_block(xb, w1_ref[0], b1_ref[0], w2_ref[0], b2_ref[0])

    @pl.when((i >= E) & ((i - E) * CAP < nfb_ref[0]))
    def _fallback():
        out_ref[...] = _ffn_block(xb, sw1_ref[...], sb1_ref[...],
                                  sw2_ref[...], sb2_ref[...])


_ffn_call = pl.pallas_call(
    _ffn_body,
    grid_spec=pltpu.PrefetchScalarGridSpec(
        num_scalar_prefetch=1,
        grid=(NSLOT // CAP,),
        in_specs=[
            pl.BlockSpec((CAP, H), lambda i, n: (i, 0)),
            pl.BlockSpec((1, H, FFN), lambda i, n: (jnp.minimum(i, E - 1), 0, 0)),
            pl.BlockSpec((1, 1, FFN), lambda i, n: (jnp.minimum(i, E - 1), 0, 0)),
            pl.BlockSpec((1, FFN, H), lambda i, n: (jnp.minimum(i, E - 1), 0, 0)),
            pl.BlockSpec((1, 1, H), lambda i, n: (jnp.minimum(i, E - 1), 0, 0)),
            pl.BlockSpec((H, FFN), lambda i, n: (0, 0)),
            pl.BlockSpec((1, FFN), lambda i, n: (0, 0)),
            pl.BlockSpec((FFN, H), lambda i, n: (0, 0)),
            pl.BlockSpec((1, H), lambda i, n: (0, 0)),
        ],
        out_specs=pl.BlockSpec((CAP, H), lambda i, n: (i, 0)),
    ),
    out_shape=jax.ShapeDtypeStruct((NSLOT, H), jnp.float32),
    compiler_params=pltpu.CompilerParams(vmem_limit_bytes=100 * 1024 * 1024),
)


def kernel(x, router_w, router_b, W1, B1, W2, B2, SW1, SB1, SW2, SB2):
    x_f = x.reshape(N, H)
    (logits, gates, idx, slot, nfb, util, f, P) = _router_call(
        x_f, router_w, router_b.reshape(1, E))
    slot_flat = slot.reshape(N)
    buf = _dispatch(x_f, slot_flat)
    out_buf = _ffn_call(nfb.reshape(1), buf,
                        W1, B1.reshape(E, 1, FFN), W2, B2.reshape(E, 1, H),
                        SW1, SB1.reshape(1, FFN), SW2, SB2.reshape(1, H))
    y_f = _combine(out_buf, slot_flat)
    return (y_f.reshape(x.shape), util.reshape(E), f.reshape(E), P.reshape(E),
            logits, gates, idx)
